# baseline (speedup 1.0000x reference)
import functools

import jax
import jax.numpy as jnp
from jax import lax
from jax.experimental import pallas as pl
from jax.experimental.pallas import tpu as pltpu
from jax.experimental.pallas import tpu_sc as plsc

B = 16384
D = 32
NUM_FIELDS = 5
P = B * NUM_FIELDS
NC, NS = 2, 16
NW = NC * NS
NH = 2
PH = P // NH
ROWS_W = PH // NW
IDX_MINOR = 128
IDX_ROWS = ROWS_W // IDX_MINOR
VOCABS = (11, 12, 31, 24, 6)
OFFSETS = (0, 11, 23, 54, 78)
V_TOTAL = 84


def _body(x_hbm, tab_hbm, out_hbm, idx_v, rows_v, tab_sh, sem, wsem):
    wid = lax.axis_index("s") * NC + lax.axis_index("c")
    base = wid * ROWS_W

    @pl.when(lax.axis_index("s") == 0)
    def _():
        pltpu.sync_copy(tab_hbm, tab_sh)

    pltpu.sync_copy(x_hbm.at[pl.ds(base, ROWS_W)], idx_v)

    lane = lax.iota(jnp.int32, 16)
    off_vecs = []
    for r in range(NUM_FIELDS):
        f = lax.rem(lane + r, jnp.int32(NUM_FIELDS))
        off = jnp.where(
            f == 1, OFFSETS[1],
            jnp.where(f == 2, OFFSETS[2],
                      jnp.where(f == 3, OFFSETS[3],
                                jnp.where(f == 4, OFFSETS[4], OFFSETS[0]))))
        off_vecs.append(off.astype(jnp.int32))

    def add_offsets(q):
        for t in range(40):
            p = q * 640 + t * 16
            v = idx_v[pl.ds(p, 16)]
            idx_v[pl.ds(p, 16)] = v + off_vecs[t % 5]

    pl.loop(0, 4)(add_offsets)

    plsc.subcore_barrier()

    copies = [
        pltpu.async_copy(
            tab_sh.at[idx_v.at[pl.ds(j * IDX_MINOR, IDX_MINOR)]],
            rows_v.at[pl.ds(j * IDX_MINOR, IDX_MINOR)],
            sem,
        )
        for j in range(IDX_ROWS)
    ]
    GRP = 5 * IDX_MINOR
    outs = []
    for g in range(IDX_ROWS // 5):
        for c in copies[g * 5:(g + 1) * 5]:
            c.wait()
        outs.append(pltpu.async_copy(
            rows_v.at[pl.ds(g * GRP, GRP)],
            out_hbm.at[pl.ds(base + g * GRP, GRP)],
            wsem,
        ))
    for o in outs:
        o.wait()


@jax.jit
def _run(x2d, fused):
    mesh = plsc.VectorSubcoreMesh(core_axis_name="c", subcore_axis_name="s")
    return pl.kernel(
        _body,
        out_type=jax.ShapeDtypeStruct((PH, D), jnp.float32),
        mesh=mesh,
        scratch_types=[
            pltpu.VMEM((ROWS_W,), jnp.int32),
            pltpu.VMEM((ROWS_W, D), jnp.float32),
            pltpu.VMEM_SHARED((V_TOTAL, D), jnp.float32),
            pltpu.SemaphoreType.DMA,
            pltpu.SemaphoreType.DMA,
        ],
        compiler_params=pltpu.CompilerParams(use_tc_tiling_on_sc=False),
    )(x2d, fused)


def kernel(x, table_year, table_month, table_day, table_hour, table_weekday):
    fused = jnp.concatenate(
        [table_year, table_month, table_day, table_hour, table_weekday],
        axis=0,
    )
    xf = x.astype(jnp.int32).reshape(-1)
    halves = [
        _run(xf[h * PH:(h + 1) * PH], fused).reshape(B // NH, NUM_FIELDS * D)
        for h in range(NH)
    ]
    return jnp.concatenate(halves, axis=0)


# speedup vs baseline: 1.2106x; 1.2106x over previous
import jax
import jax.numpy as jnp
from jax import lax
from jax.experimental import pallas as pl
from jax.experimental.pallas import tpu as pltpu
from jax.experimental.pallas import tpu_sc as plsc

B = 16384
D = 32
NUM_FIELDS = 5
P = B * NUM_FIELDS
NC, NS = 2, 16
NW = NC * NS
ROWS_W = P // NW
IDX_MINOR = 128
IDX_ROWS = ROWS_W // IDX_MINOR
OFFSETS = (0, 11, 23, 54, 78)
V_TOTAL = 84


def _body(x_hbm, tab_hbm, out_hbm, idx_v, rows_v, tab_sh, sem, wsem):
    wid = lax.axis_index("s") * NC + lax.axis_index("c")
    base = wid * ROWS_W

    @pl.when(lax.axis_index("s") == 0)
    def _():
        pltpu.sync_copy(tab_hbm, tab_sh)

    pltpu.sync_copy(x_hbm.at[pl.ds(base, ROWS_W)], idx_v)

    lane = lax.iota(jnp.int32, 16)
    off_vecs = []
    for r in range(NUM_FIELDS):
        f = lax.rem(lane + r, jnp.int32(NUM_FIELDS))
        off = jnp.where(
            f == 1, OFFSETS[1],
            jnp.where(f == 2, OFFSETS[2],
                      jnp.where(f == 3, OFFSETS[3],
                                jnp.where(f == 4, OFFSETS[4], OFFSETS[0]))))
        off_vecs.append(off.astype(jnp.int32))

    def add_offsets(q):
        for t in range(40):
            p = q * 640 + t * 16
            v = idx_v[pl.ds(p, 16)]
            idx_v[pl.ds(p, 16)] = v + off_vecs[t % 5]

    pl.loop(0, 4)(add_offsets)

    plsc.subcore_barrier()

    copies = [
        pltpu.async_copy(
            tab_sh.at[idx_v.at[pl.ds(j * IDX_MINOR, IDX_MINOR)]],
            rows_v.at[pl.ds(j * IDX_MINOR, IDX_MINOR)],
            sem,
        )
        for j in range(IDX_ROWS)
    ]
    GRP = 5 * IDX_MINOR
    outs = []
    for g in range(IDX_ROWS // 5):
        for c in copies[g * 5:(g + 1) * 5]:
            c.wait()
        outs.append(pltpu.async_copy(
            rows_v.at[pl.ds(g * GRP, GRP)],
            out_hbm.at[pl.ds(base + g * GRP, GRP)],
            wsem,
        ))
    for o in outs:
        o.wait()


@jax.jit
def _run(x2d, fused):
    mesh = plsc.VectorSubcoreMesh(core_axis_name="c", subcore_axis_name="s")
    return pl.kernel(
        _body,
        out_type=jax.ShapeDtypeStruct((P, D), jnp.float32),
        mesh=mesh,
        scratch_types=[
            pltpu.VMEM((ROWS_W,), jnp.int32),
            pltpu.VMEM((ROWS_W, D), jnp.float32),
            pltpu.VMEM_SHARED((V_TOTAL, D), jnp.float32),
            pltpu.SemaphoreType.DMA,
            pltpu.SemaphoreType.DMA,
        ],
        compiler_params=pltpu.CompilerParams(use_tc_tiling_on_sc=False),
    )(x2d, fused)


def kernel(x, table_year, table_month, table_day, table_hour, table_weekday):
    fused = jnp.concatenate(
        [table_year, table_month, table_day, table_hour, table_weekday],
        axis=0,
    )
    out = _run(x.astype(jnp.int32).reshape(-1), fused)
    return out.reshape(B, NUM_FIELDS * D)


# speedup vs baseline: 1.2162x; 1.0046x over previous
import jax
import jax.numpy as jnp
from jax import lax
from jax.experimental import pallas as pl
from jax.experimental.pallas import tpu as pltpu
from jax.experimental.pallas import tpu_sc as plsc

B = 16384
D = 32
NUM_FIELDS = 5
P = B * NUM_FIELDS
NC, NS = 2, 16
NW = NC * NS
ROWS_W = P // NW
IDX_MINOR = 512
IDX_ROWS = ROWS_W // IDX_MINOR
NDRAIN = 1
OFFSETS = (0, 11, 23, 54, 78)
V_TOTAL = 84


def _body(x_hbm, tab_hbm, out_hbm, idx_v, rows_v, tab_sh, sem, wsem):
    wid = lax.axis_index("s") * NC + lax.axis_index("c")
    base = wid * ROWS_W

    @pl.when(lax.axis_index("s") == 0)
    def _():
        pltpu.sync_copy(tab_hbm, tab_sh)

    pltpu.sync_copy(x_hbm.at[pl.ds(base, ROWS_W)], idx_v)

    lane = lax.iota(jnp.int32, 16)
    off_vecs = []
    for r in range(NUM_FIELDS):
        f = lax.rem(lane + r, jnp.int32(NUM_FIELDS))
        off = jnp.where(
            f == 1, OFFSETS[1],
            jnp.where(f == 2, OFFSETS[2],
                      jnp.where(f == 3, OFFSETS[3],
                                jnp.where(f == 4, OFFSETS[4], OFFSETS[0]))))
        off_vecs.append(off.astype(jnp.int32))

    def add_offsets(q):
        for t in range(40):
            p = q * 640 + t * 16
            v = idx_v[pl.ds(p, 16)]
            idx_v[pl.ds(p, 16)] = v + off_vecs[t % 5]

    pl.loop(0, 4)(add_offsets)

    plsc.subcore_barrier()

    copies = [
        pltpu.async_copy(
            tab_sh.at[idx_v.at[pl.ds(j * IDX_MINOR, IDX_MINOR)]],
            rows_v.at[pl.ds(j * IDX_MINOR, IDX_MINOR)],
            sem,
        )
        for j in range(IDX_ROWS)
    ]
    GRP = NDRAIN * IDX_MINOR
    outs = []
    for g in range(IDX_ROWS // NDRAIN):
        for c in copies[g * NDRAIN:(g + 1) * NDRAIN]:
            c.wait()
        outs.append(pltpu.async_copy(
            rows_v.at[pl.ds(g * GRP, GRP)],
            out_hbm.at[pl.ds(base + g * GRP, GRP)],
            wsem,
        ))
    for o in outs:
        o.wait()


@jax.jit
def _run(x2d, fused):
    mesh = plsc.VectorSubcoreMesh(core_axis_name="c", subcore_axis_name="s")
    return pl.kernel(
        _body,
        out_type=jax.ShapeDtypeStruct((P, D), jnp.float32),
        mesh=mesh,
        scratch_types=[
            pltpu.VMEM((ROWS_W,), jnp.int32),
            pltpu.VMEM((ROWS_W, D), jnp.float32),
            pltpu.VMEM_SHARED((V_TOTAL, D), jnp.float32),
            pltpu.SemaphoreType.DMA,
            pltpu.SemaphoreType.DMA,
        ],
        compiler_params=pltpu.CompilerParams(use_tc_tiling_on_sc=False),
    )(x2d, fused)


def kernel(x, table_year, table_month, table_day, table_hour, table_weekday):
    fused = jnp.concatenate(
        [table_year, table_month, table_day, table_hour, table_weekday],
        axis=0,
    )
    out = _run(x.astype(jnp.int32).reshape(-1), fused)
    return out.reshape(B, NUM_FIELDS * D)
